# Initial kernel scaffold; baseline (speedup 1.0000x reference)
import jax
import jax.numpy as jnp
from jax.experimental import pallas as pl


def kernel(x, edge_index, W):
    raise NotImplementedError("write your pallas kernel here")


# speedup vs baseline: 14.7694x; 14.7694x over previous
import functools

import jax
import jax.numpy as jnp
from jax import lax
from jax.experimental import pallas as pl
from jax.experimental.pallas import tpu as pltpu
from jax.experimental.pallas import tpu_sc as plsc

N = 10000
E = 320000
D = 128
C = 40
DP = 48
NP = 10240
NC = 2
NS = 16
NW = NC * NS
EPT = E // NW
B = 80
NCH = EPT // B
RPS = NP // NS

_MESH = plsc.VectorSubcoreMesh(core_axis_name="c", subcore_axis_name="s")


def _edge_scatter_body(do_gather, g_hbm, src_hbm, dst_hbm, z_hbm, out_hbm,
                       acc, src_v, dst_v, rows_v, sem):
    cid = lax.axis_index("c")
    sid = lax.axis_index("s")
    w = cid * NS + sid

    pltpu.sync_copy(z_hbm.at[pl.ds(sid * RPS, RPS)],
                    acc.at[pl.ds(sid * RPS, RPS)])

    if not do_gather:
        pltpu.sync_copy(g_hbm.at[pl.ds(0, B)], rows_v)

    plsc.subcore_barrier()

    ebase = w * EPT

    def chunk(j, carry):
        off = ebase + j * B
        pltpu.sync_copy(dst_hbm.at[pl.ds(off, B)], dst_v)
        if do_gather:
            pltpu.sync_copy(src_hbm.at[pl.ds(off, B)], src_v)
            pltpu.async_copy(g_hbm.at[src_v], rows_v, sem).wait()
        pltpu.sync_copy(rows_v, acc.at[dst_v], add=True)
        return carry

    lax.fori_loop(0, NCH, chunk, 0)

    plsc.subcore_barrier()

    pltpu.sync_copy(acc.at[pl.ds(sid * RPS, RPS)],
                    out_hbm.at[cid, pl.ds(sid * RPS, RPS)])


def _make_edge_scatter(do_gather):
    return pl.kernel(
        functools.partial(_edge_scatter_body, do_gather),
        out_type=jax.ShapeDtypeStruct((NC, NP, DP), jnp.float32),
        mesh=_MESH,
        scratch_types=[
            pltpu.VMEM_SHARED((NP, DP), jnp.float32),
            pltpu.VMEM((B,), jnp.int32),
            pltpu.VMEM((B,), jnp.int32),
            pltpu.VMEM((B, DP), jnp.float32),
            pltpu.SemaphoreType.DMA,
        ],
        compiler_params=pltpu.CompilerParams(use_tc_tiling_on_sc=False),
    )


_edge_scatter = _make_edge_scatter(True)
_deg_scatter = _make_edge_scatter(False)


def _dense1_body(x_ref, w_ref, parts_ref, dinv_ref, g1_ref):
    y = jnp.dot(x_ref[...], w_ref[...], preferred_element_type=jnp.float32)
    cnt = parts_ref[0, :, 0:1] + parts_ref[1, :, 0:1]
    dinv = lax.rsqrt(cnt + 1.0)
    dinv_ref[...] = dinv
    g1_ref[...] = y * dinv


def _dense2_body(square, parts_ref, g_ref, dinv_ref, out_ref):
    s = parts_ref[0] + parts_ref[1] + g_ref[...]
    d = dinv_ref[...]
    scale = d * d if square else d
    out_ref[...] = s * scale


_dense1 = pl.pallas_call(
    _dense1_body,
    out_shape=(jax.ShapeDtypeStruct((NP, 1), jnp.float32),
               jax.ShapeDtypeStruct((NP, DP), jnp.float32)),
)

_dense2a = pl.pallas_call(
    functools.partial(_dense2_body, True),
    out_shape=jax.ShapeDtypeStruct((NP, DP), jnp.float32),
)

_dense2b = pl.pallas_call(
    functools.partial(_dense2_body, False),
    out_shape=jax.ShapeDtypeStruct((NP, DP), jnp.float32),
)


@jax.jit
def kernel(x, edge_index, W):
    src = edge_index[0].astype(jnp.int32)
    dst = edge_index[1].astype(jnp.int32)
    xp = jnp.pad(x, ((0, NP - N), (0, 0)))
    Wp = jnp.pad(W, ((0, 0), (0, DP - C)))
    zeros = jnp.zeros((NP, DP), jnp.float32)
    ones = jnp.ones((NP, DP), jnp.float32)

    deg_parts = _deg_scatter(ones, src, dst, zeros)
    dinv, g1 = _dense1(xp, Wp, deg_parts)
    parts1 = _edge_scatter(g1, src, dst, zeros)
    g2 = _dense2a(parts1, g1, dinv)
    parts2 = _edge_scatter(g2, src, dst, zeros)
    outp = _dense2b(parts2, g2, dinv)
    return outp[:N, :C]


# speedup vs baseline: 18.9975x; 1.2863x over previous
import functools

import jax
import jax.numpy as jnp
from jax import lax
from jax.experimental import pallas as pl
from jax.experimental.pallas import tpu as pltpu
from jax.experimental.pallas import tpu_sc as plsc

N = 10000
E = 320000
D = 128
C = 40
DP = 48
NP = 10240
NC = 2
NS = 16
NW = NC * NS
B = 128
NCH = 80
EPT = NCH * B
EP = NW * EPT
RPS = NP // NS
NBUF = 4
DEG_LAG = 8

_MESH = plsc.VectorSubcoreMesh(core_axis_name="c", subcore_axis_name="s")
_SC_PARAMS = pltpu.CompilerParams(use_tc_tiling_on_sc=False)


def _hop_body(g_hbm, src_hbm, dst_hbm, z_hbm, out_hbm,
              acc, src_all, dst_all, r0, r1, r2, r3,
              g0, g1, g2, g3, s0, s1, s2, s3):
    cid = lax.axis_index("c")
    sid = lax.axis_index("s")
    w = cid * NS + sid
    rows = (r0, r1, r2, r3)
    gsem = (g0, g1, g2, g3)
    ssem = (s0, s1, s2, s3)

    pltpu.sync_copy(z_hbm.at[pl.ds(sid * RPS, RPS)],
                    acc.at[pl.ds(sid * RPS, RPS)])
    pltpu.sync_copy(src_hbm.at[w], src_all)
    pltpu.sync_copy(dst_hbm.at[w], dst_all)
    plsc.subcore_barrier()

    def gissue(b, j):
        pltpu.async_copy(g_hbm.at[src_all.at[j]], rows[b], gsem[b])

    def gwait(b):
        pltpu.make_async_copy(g_hbm.at[src_all.at[0]], rows[b], gsem[b]).wait()

    def sissue(b, j):
        pltpu.async_copy(rows[b], acc.at[dst_all.at[j]], ssem[b], add=True)

    def swait(b):
        pltpu.make_async_copy(rows[b], acc.at[dst_all.at[0]], ssem[b]).wait()

    gissue(0, 0)
    gissue(1, 1)
    for j in (0, 1):
        gwait(j)
        sissue(j, j)
        gissue(j + 2, j + 2)

    def grp(g, carry):
        for bi in range(NBUF):
            j = 2 + g * NBUF + bi
            gwait((2 + bi) % NBUF)
            sissue((2 + bi) % NBUF, j)
            swait(bi)
            gissue(bi, j + 2)
        return carry

    lax.fori_loop(0, (NCH - 4) // NBUF, grp, 0)

    for j in (NCH - 2, NCH - 1):
        gwait(j % NBUF)
        sissue(j % NBUF, j)
        swait((j - 2) % NBUF)
    swait((NCH - 2) % NBUF)
    swait((NCH - 1) % NBUF)

    plsc.subcore_barrier()
    pltpu.sync_copy(acc.at[pl.ds(sid * RPS, RPS)],
                    out_hbm.at[cid, pl.ds(sid * RPS, RPS)])


_hop = pl.kernel(
    _hop_body,
    out_type=jax.ShapeDtypeStruct((NC, NP, DP), jnp.float32),
    mesh=_MESH,
    scratch_types=[
        pltpu.VMEM_SHARED((NP, DP), jnp.float32),
        pltpu.VMEM((NCH, B), jnp.int32),
        pltpu.VMEM((NCH, B), jnp.int32),
        pltpu.VMEM((B, DP), jnp.float32),
        pltpu.VMEM((B, DP), jnp.float32),
        pltpu.VMEM((B, DP), jnp.float32),
        pltpu.VMEM((B, DP), jnp.float32),
        pltpu.SemaphoreType.DMA,
        pltpu.SemaphoreType.DMA,
        pltpu.SemaphoreType.DMA,
        pltpu.SemaphoreType.DMA,
        pltpu.SemaphoreType.DMA,
        pltpu.SemaphoreType.DMA,
        pltpu.SemaphoreType.DMA,
        pltpu.SemaphoreType.DMA,
    ],
    compiler_params=_SC_PARAMS,
)

DEGW = 16


def _deg_body(ones_hbm, dst_hbm, z_hbm, out_hbm, acc, dst_all, ones_v, sem):
    cid = lax.axis_index("c")
    sid = lax.axis_index("s")
    w = cid * NS + sid

    pltpu.sync_copy(z_hbm.at[pl.ds(sid * RPS, RPS)],
                    acc.at[pl.ds(sid * RPS, RPS)])
    pltpu.sync_copy(dst_hbm.at[w], dst_all)
    pltpu.sync_copy(ones_hbm, ones_v)
    plsc.subcore_barrier()

    def issue(j):
        pltpu.async_copy(ones_v, acc.at[dst_all.at[j]], sem, add=True)

    def drain_one():
        pltpu.make_async_copy(ones_v, acc.at[dst_all.at[0]], sem).wait()

    for j in range(DEG_LAG):
        issue(j)

    def step(j, carry):
        issue(j)
        drain_one()
        return carry

    lax.fori_loop(DEG_LAG, NCH, step, 0)
    for _ in range(DEG_LAG):
        drain_one()

    plsc.subcore_barrier()
    pltpu.sync_copy(acc.at[pl.ds(sid * RPS, RPS)],
                    out_hbm.at[cid, pl.ds(sid * RPS, RPS)])


_deg = pl.kernel(
    _deg_body,
    out_type=jax.ShapeDtypeStruct((NC, NP, DEGW), jnp.float32),
    mesh=_MESH,
    scratch_types=[
        pltpu.VMEM_SHARED((NP, DEGW), jnp.float32),
        pltpu.VMEM((NCH, B), jnp.int32),
        pltpu.VMEM((B, DEGW), jnp.float32),
        pltpu.SemaphoreType.DMA,
    ],
    compiler_params=_SC_PARAMS,
)


def _mm_body(x_ref, w_ref, y_ref):
    y_ref[...] = jnp.dot(x_ref[...], w_ref[...],
                         preferred_element_type=jnp.float32)


def _scale1_body(parts_ref, y_ref, dinv_ref, g1_ref):
    cnt = parts_ref[0, :, 0:1] + parts_ref[1, :, 0:1]
    dinv = lax.rsqrt(cnt + 1.0)
    dinv_ref[...] = dinv
    g1_ref[...] = y_ref[...] * dinv


def _dense2_body(square, parts_ref, g_ref, dinv_ref, out_ref):
    s = parts_ref[0] + parts_ref[1] + g_ref[...]
    d = dinv_ref[...]
    scale = d * d if square else d
    out_ref[...] = s * scale


_mm = pl.pallas_call(
    _mm_body,
    out_shape=jax.ShapeDtypeStruct((NP, DP), jnp.float32),
)

_scale1 = pl.pallas_call(
    _scale1_body,
    out_shape=(jax.ShapeDtypeStruct((NP, 1), jnp.float32),
               jax.ShapeDtypeStruct((NP, DP), jnp.float32)),
)

_dense2a = pl.pallas_call(
    functools.partial(_dense2_body, True),
    out_shape=jax.ShapeDtypeStruct((NP, DP), jnp.float32),
)

_dense2b = pl.pallas_call(
    functools.partial(_dense2_body, False),
    out_shape=jax.ShapeDtypeStruct((NP, DP), jnp.float32),
)


@jax.jit
def kernel(x, edge_index, W):
    pad = jnp.full((EP - E,), N, jnp.int32)
    src = jnp.concatenate([edge_index[0].astype(jnp.int32), pad])
    dst = jnp.concatenate([edge_index[1].astype(jnp.int32), pad])
    src_r = src.reshape(NW, NCH, B)
    dst_r = dst.reshape(NW, NCH, B)

    xp = jnp.pad(x, ((0, NP - N), (0, 0)))
    Wp = jnp.pad(W, ((0, 0), (0, DP - C)))
    zeros = jnp.zeros((NP, DP), jnp.float32)
    zeros16 = jnp.zeros((NP, DEGW), jnp.float32)
    ones16 = jnp.ones((B, DEGW), jnp.float32)

    deg_parts = _deg(ones16, dst_r, zeros16)
    y = _mm(xp, Wp)
    dinv, g1 = _scale1(deg_parts, y)
    parts1 = _hop(g1, src_r, dst_r, zeros)
    g2 = _dense2a(parts1, g1, dinv)
    parts2 = _hop(g2, src_r, dst_r, zeros)
    outp = _dense2b(parts2, g2, dinv)
    return outp[:N, :C]


# speedup vs baseline: 42.5516x; 2.2399x over previous
import functools

import jax
import jax.numpy as jnp
from jax import lax
from jax.experimental import pallas as pl
from jax.experimental.pallas import tpu as pltpu
from jax.experimental.pallas import tpu_sc as plsc

N = 10000
E = 320000
D = 128
C = 40
DP = 48
NP = 10240
NC = 2
NS = 16
NW = NC * NS
B = 128
NCH = 80
EPT = NCH * B
EP = NW * EPT
RPS = NP // NS
NBUF = 4
DEG_LAG = 8

_MESH = plsc.VectorSubcoreMesh(core_axis_name="c", subcore_axis_name="s")
_SC_PARAMS = pltpu.CompilerParams(use_tc_tiling_on_sc=False)


def _hop_body(g_hbm, src_hbm, dst_hbm, z_hbm, out_hbm,
              acc, src_all, dst_all, r0, r1, r2, r3,
              g0, g1, g2, g3, s0, s1, s2, s3):
    cid = lax.axis_index("c")
    sid = lax.axis_index("s")
    w = cid * NS + sid
    rows = (r0, r1, r2, r3)
    gsem = (g0, g1, g2, g3)
    ssem = (s0, s1, s2, s3)

    pltpu.sync_copy(z_hbm.at[pl.ds(sid * RPS, RPS)],
                    acc.at[pl.ds(sid * RPS, RPS)])
    pltpu.sync_copy(src_hbm.at[w], src_all)
    pltpu.sync_copy(dst_hbm.at[w], dst_all)
    plsc.subcore_barrier()

    def gissue(b, j):
        pltpu.async_copy(g_hbm.at[src_all.at[j]], rows[b], gsem[b])

    def gwait(b):
        pltpu.make_async_copy(g_hbm.at[src_all.at[0]], rows[b], gsem[b]).wait()

    def sissue(b, j):
        pltpu.async_copy(rows[b], acc.at[dst_all.at[j]], ssem[b], add=True)

    def swait(b):
        pltpu.make_async_copy(rows[b], acc.at[dst_all.at[0]], ssem[b]).wait()

    gissue(0, 0)
    gissue(1, 1)
    for j in (0, 1):
        gwait(j)
        sissue(j, j)
        gissue(j + 2, j + 2)

    def grp(g, carry):
        for bi in range(NBUF):
            j = 2 + g * NBUF + bi
            gwait((2 + bi) % NBUF)
            sissue((2 + bi) % NBUF, j)
            swait(bi)
            gissue(bi, j + 2)
        return carry

    lax.fori_loop(0, (NCH - 4) // NBUF, grp, 0)

    for j in (NCH - 2, NCH - 1):
        gwait(j % NBUF)
        sissue(j % NBUF, j)
        swait((j - 2) % NBUF)
    swait((NCH - 2) % NBUF)
    swait((NCH - 1) % NBUF)

    plsc.subcore_barrier()
    pltpu.sync_copy(acc.at[pl.ds(sid * RPS, RPS)],
                    out_hbm.at[cid, pl.ds(sid * RPS, RPS)])


_hop = pl.kernel(
    _hop_body,
    out_type=jax.ShapeDtypeStruct((NC, NP, DP), jnp.float32),
    mesh=_MESH,
    scratch_types=[
        pltpu.VMEM_SHARED((NP, DP), jnp.float32),
        pltpu.VMEM((NCH, B), jnp.int32),
        pltpu.VMEM((NCH, B), jnp.int32),
        pltpu.VMEM((B, DP), jnp.float32),
        pltpu.VMEM((B, DP), jnp.float32),
        pltpu.VMEM((B, DP), jnp.float32),
        pltpu.VMEM((B, DP), jnp.float32),
        pltpu.SemaphoreType.DMA,
        pltpu.SemaphoreType.DMA,
        pltpu.SemaphoreType.DMA,
        pltpu.SemaphoreType.DMA,
        pltpu.SemaphoreType.DMA,
        pltpu.SemaphoreType.DMA,
        pltpu.SemaphoreType.DMA,
        pltpu.SemaphoreType.DMA,
    ],
    compiler_params=_SC_PARAMS,
)

DEGW = 16


def _deg_body(ones_hbm, dst_hbm, z_hbm, out_hbm, acc, dst_all, ones_v, sem):
    cid = lax.axis_index("c")
    sid = lax.axis_index("s")
    w = cid * NS + sid

    pltpu.sync_copy(z_hbm.at[pl.ds(sid * RPS, RPS)],
                    acc.at[pl.ds(sid * RPS, RPS)])
    pltpu.sync_copy(dst_hbm.at[w], dst_all)
    pltpu.sync_copy(ones_hbm, ones_v)
    plsc.subcore_barrier()

    def issue(j):
        pltpu.async_copy(ones_v, acc.at[dst_all.at[j]], sem, add=True)

    def drain_one():
        pltpu.make_async_copy(ones_v, acc.at[dst_all.at[0]], sem).wait()

    for j in range(DEG_LAG):
        issue(j)

    def step(j, carry):
        issue(j)
        drain_one()
        return carry

    lax.fori_loop(DEG_LAG, NCH, step, 0)
    for _ in range(DEG_LAG):
        drain_one()

    plsc.subcore_barrier()
    pltpu.sync_copy(acc.at[pl.ds(sid * RPS, RPS)],
                    out_hbm.at[cid, pl.ds(sid * RPS, RPS)])


_deg = pl.kernel(
    _deg_body,
    out_type=jax.ShapeDtypeStruct((NC, NP, DEGW), jnp.float32),
    mesh=_MESH,
    scratch_types=[
        pltpu.VMEM_SHARED((NP, DEGW), jnp.float32),
        pltpu.VMEM((NCH, B), jnp.int32),
        pltpu.VMEM((B, DEGW), jnp.float32),
        pltpu.SemaphoreType.DMA,
    ],
    compiler_params=_SC_PARAMS,
)


def _mm_body(x_ref, w_ref, y_ref):
    y_ref[...] = jnp.dot(x_ref[...], w_ref[...],
                         preferred_element_type=jnp.float32)


def _scale1_body(parts_ref, y_ref, dinv_ref, g1_ref):
    cnt = parts_ref[0, :, 0:1] + parts_ref[1, :, 0:1]
    dinv = lax.rsqrt(cnt + 1.0)
    dinv_ref[...] = dinv
    g1_ref[...] = y_ref[...] * dinv


def _dense2_body(square, parts_ref, g_ref, dinv_ref, out_ref):
    s = parts_ref[0] + parts_ref[1] + g_ref[...]
    d = dinv_ref[...]
    scale = d * d if square else d
    out_ref[...] = s * scale


_mm = pl.pallas_call(
    _mm_body,
    out_shape=jax.ShapeDtypeStruct((NP, DP), jnp.float32),
)

_scale1 = pl.pallas_call(
    _scale1_body,
    out_shape=(jax.ShapeDtypeStruct((NP, 1), jnp.float32),
               jax.ShapeDtypeStruct((NP, DP), jnp.float32)),
)

_dense2a = pl.pallas_call(
    functools.partial(_dense2_body, True),
    out_shape=jax.ShapeDtypeStruct((NP, DP), jnp.float32),
)

_dense2b = pl.pallas_call(
    functools.partial(_dense2_body, False),
    out_shape=jax.ShapeDtypeStruct((NP, DP), jnp.float32),
)


@jax.jit
def kernel(x, edge_index, W):
    pad = N + (jnp.arange(EP - E, dtype=jnp.int32) % (NP - N))
    src = jnp.concatenate([edge_index[0].astype(jnp.int32), pad])
    dst = jnp.concatenate([edge_index[1].astype(jnp.int32), pad])
    src_r = src.reshape(NW, NCH, B)
    dst_r = dst.reshape(NW, NCH, B)

    xp = jnp.pad(x, ((0, NP - N), (0, 0)))
    Wp = jnp.pad(W, ((0, 0), (0, DP - C)))
    zeros = jnp.zeros((NP, DP), jnp.float32)
    zeros16 = jnp.zeros((NP, DEGW), jnp.float32)
    ones16 = jnp.ones((B, DEGW), jnp.float32)

    deg_parts = _deg(ones16, dst_r, zeros16)
    y = _mm(xp, Wp)
    dinv, g1 = _scale1(deg_parts, y)
    parts1 = _hop(g1, src_r, dst_r, zeros)
    g2 = _dense2a(parts1, g1, dinv)
    parts2 = _hop(g2, src_r, dst_r, zeros)
    outp = _dense2b(parts2, g2, dinv)
    return outp[:N, :C]


# speedup vs baseline: 45.3713x; 1.0663x over previous
import functools

import jax
import jax.numpy as jnp
from jax import lax
from jax.experimental import pallas as pl
from jax.experimental.pallas import tpu as pltpu
from jax.experimental.pallas import tpu_sc as plsc

N = 10000
E = 320000
D = 128
C = 40
DP = 48
NP = 10240
NC = 2
NS = 16
NW = NC * NS
B = 80
NCH = 125
RPS = NP // NS
NBUF = 8
K = NBUF // 2
DEG_LAG = 8

_MESH = plsc.VectorSubcoreMesh(core_axis_name="c", subcore_axis_name="s")
_SC_PARAMS = pltpu.CompilerParams(use_tc_tiling_on_sc=False)


def _hop_body(g_hbm, src_hbm, dst_hbm, z_hbm, out_hbm,
              acc, src_all, dst_all, rows, gsem, ssem):
    cid = lax.axis_index("c")
    sid = lax.axis_index("s")
    w = cid * NS + sid

    pltpu.sync_copy(z_hbm.at[pl.ds(sid * RPS, RPS)],
                    acc.at[pl.ds(sid * RPS, RPS)])
    pltpu.sync_copy(src_hbm.at[w], src_all)
    pltpu.sync_copy(dst_hbm.at[w], dst_all)
    plsc.subcore_barrier()

    def gissue(b, j):
        pltpu.async_copy(g_hbm.at[src_all.at[j]], rows[b], gsem[b])

    def gwait(b):
        pltpu.make_async_copy(g_hbm.at[src_all.at[0]], rows[b], gsem[b]).wait()

    def sissue(b, j):
        pltpu.async_copy(rows[b], acc.at[dst_all.at[j]], ssem[b], add=True)

    def swait(b):
        pltpu.make_async_copy(rows[b], acc.at[dst_all.at[0]], ssem[b]).wait()

    for j in range(K):
        gissue(j % NBUF, j)
    for j in range(K):
        gwait(j % NBUF)
        sissue(j % NBUF, j)
        gissue((j + K) % NBUF, j + K)

    n_grp = (NCH - 2 * K) // NBUF

    def grp(g, carry):
        for bi in range(NBUF):
            j = K + g * NBUF + bi
            b = (K + bi) % NBUF
            gwait(b)
            sissue(b, j)
            swait(bi)
            gissue(bi, j + K)
        return carry

    lax.fori_loop(0, n_grp, grp, 0)

    for j in range(K + n_grp * NBUF, NCH - K):
        gwait(j % NBUF)
        sissue(j % NBUF, j)
        swait((j - K) % NBUF)
        gissue((j + K) % NBUF, j + K)
    for j in range(NCH - K, NCH):
        gwait(j % NBUF)
        sissue(j % NBUF, j)
        swait((j - K) % NBUF)
    for j in range(NCH - K, NCH):
        swait(j % NBUF)

    plsc.subcore_barrier()

    pltpu.sync_copy(acc.at[pl.ds(sid * RPS, RPS)],
                    out_hbm.at[cid, pl.ds(sid * RPS, RPS)])


_hop = pl.kernel(
    lambda g, s, d, z, o, acc, sa, da, *bufs: _hop_body(
        g, s, d, z, o, acc, sa, da,
        bufs[:NBUF], bufs[NBUF:2 * NBUF], bufs[2 * NBUF:]),
    out_type=jax.ShapeDtypeStruct((NC, NP, DP), jnp.float32),
    mesh=_MESH,
    scratch_types=[
        pltpu.VMEM_SHARED((NP, DP), jnp.float32),
        pltpu.VMEM((NCH, B), jnp.int32),
        pltpu.VMEM((NCH, B), jnp.int32),
    ] + [pltpu.VMEM((B, DP), jnp.float32)] * NBUF
      + [pltpu.SemaphoreType.DMA] * (2 * NBUF),
    compiler_params=_SC_PARAMS,
)

DEGW = 16


def _deg_body(ones_hbm, dst_hbm, z_hbm, out_hbm, acc, dst_all, ones_v, sem):
    cid = lax.axis_index("c")
    sid = lax.axis_index("s")
    w = cid * NS + sid

    pltpu.sync_copy(z_hbm.at[pl.ds(sid * RPS, RPS)],
                    acc.at[pl.ds(sid * RPS, RPS)])
    pltpu.sync_copy(dst_hbm.at[w], dst_all)
    pltpu.sync_copy(ones_hbm, ones_v)
    plsc.subcore_barrier()

    def issue(j):
        pltpu.async_copy(ones_v, acc.at[dst_all.at[j]], sem, add=True)

    def drain_one():
        pltpu.make_async_copy(ones_v, acc.at[dst_all.at[0]], sem).wait()

    for j in range(DEG_LAG):
        issue(j)

    def step(j, carry):
        issue(j)
        drain_one()
        return carry

    lax.fori_loop(DEG_LAG, NCH, step, 0)
    for _ in range(DEG_LAG):
        drain_one()

    plsc.subcore_barrier()
    pltpu.sync_copy(acc.at[pl.ds(sid * RPS, RPS)],
                    out_hbm.at[cid, pl.ds(sid * RPS, RPS)])


_deg = pl.kernel(
    _deg_body,
    out_type=jax.ShapeDtypeStruct((NC, NP, DEGW), jnp.float32),
    mesh=_MESH,
    scratch_types=[
        pltpu.VMEM_SHARED((NP, DEGW), jnp.float32),
        pltpu.VMEM((NCH, B), jnp.int32),
        pltpu.VMEM((B, DEGW), jnp.float32),
        pltpu.SemaphoreType.DMA,
    ],
    compiler_params=_SC_PARAMS,
)


def _mm_body(x_ref, w_ref, y_ref):
    y_ref[...] = jnp.dot(x_ref[...], w_ref[...],
                         preferred_element_type=jnp.float32)


def _scale1_body(parts_ref, y_ref, dinv_ref, g1_ref):
    cnt = parts_ref[0, :, 0:1] + parts_ref[1, :, 0:1]
    dinv = lax.rsqrt(cnt + 1.0)
    dinv_ref[...] = dinv
    g1_ref[...] = y_ref[...] * dinv


def _dense2_body(square, parts_ref, g_ref, dinv_ref, out_ref):
    s = parts_ref[0] + parts_ref[1] + g_ref[...]
    d = dinv_ref[...]
    scale = d * d if square else d
    out_ref[...] = s * scale


_mm = pl.pallas_call(
    _mm_body,
    out_shape=jax.ShapeDtypeStruct((NP, DP), jnp.float32),
)

_scale1 = pl.pallas_call(
    _scale1_body,
    out_shape=(jax.ShapeDtypeStruct((NP, 1), jnp.float32),
               jax.ShapeDtypeStruct((NP, DP), jnp.float32)),
)

_dense2a = pl.pallas_call(
    functools.partial(_dense2_body, True),
    out_shape=jax.ShapeDtypeStruct((NP, DP), jnp.float32),
)

_dense2b = pl.pallas_call(
    functools.partial(_dense2_body, False),
    out_shape=jax.ShapeDtypeStruct((NP, DP), jnp.float32),
)


@jax.jit
def kernel(x, edge_index, W):
    src_r = edge_index[0].astype(jnp.int32).reshape(NW, NCH, B)
    dst_r = edge_index[1].astype(jnp.int32).reshape(NW, NCH, B)

    xp = jnp.pad(x, ((0, NP - N), (0, 0)))
    Wp = jnp.pad(W, ((0, 0), (0, DP - C)))
    zeros = jnp.zeros((NP, DP), jnp.float32)
    zeros16 = jnp.zeros((NP, DEGW), jnp.float32)
    ones16 = jnp.ones((B, DEGW), jnp.float32)

    deg_parts = _deg(ones16, dst_r, zeros16)
    y = _mm(xp, Wp)
    dinv, g1 = _scale1(deg_parts, y)
    parts1 = _hop(g1, src_r, dst_r, zeros)
    g2 = _dense2a(parts1, g1, dinv)
    parts2 = _hop(g2, src_r, dst_r, zeros)
    outp = _dense2b(parts2, g2, dinv)
    return outp[:N, :C]
